# baseline (speedup 1.0000x reference)
import functools

import jax
import jax.numpy as jnp
from jax import lax
from jax.experimental import pallas as pl
from jax.experimental.pallas import tpu as pltpu
from jax.experimental.pallas import tpu_sc as plsc

_NC = 2
_NS = 16
_NW = _NC * _NS
_L = 16
_K = 128

_N = 500000
_U = 8
_CH = 15744


@functools.partial(
    pl.kernel,
    out_type=jax.ShapeDtypeStruct((_N,), jnp.int32),
    mesh=plsc.VectorSubcoreMesh(core_axis_name="c", subcore_axis_name="s"),
    compiler_params=pltpu.CompilerParams(needs_layout_passes=False),
    scratch_types=[
        pltpu.VMEM((_CH,), jnp.int32),
        pltpu.VMEM((_CH,), jnp.int32),
        pltpu.VMEM((_K,), jnp.int32),
        pltpu.VMEM((_K,), jnp.int32),
        pltpu.VMEM((_K,), jnp.int32),
    ],
)
def _sc_lookup(x_hbm, keys_hbm, out_hbm, xv, ov, kv, sk, sl):
    wid = lax.axis_index("s") * _NC + lax.axis_index("c")
    base = jnp.minimum(wid * _CH, _N - _CH)
    pltpu.sync_copy(keys_hbm, kv)
    pltpu.sync_copy(x_hbm.at[pl.ds(base, _CH)], xv)

    kvec = [kv[pl.ds(i * _L, _L)] for i in range(_K // _L)]

    def rank_body(j, ranks):
        bc = plsc.load_gather(kv, [jnp.full((_L,), j, jnp.int32)])
        return tuple(
            r + jnp.where(bc < kvec[i], 1, 0) for i, r in enumerate(ranks)
        )

    zero = jnp.zeros((_L,), jnp.int32)
    ranks = lax.fori_loop(0, _K, rank_body, (zero,) * (_K // _L))
    lane = lax.iota(jnp.int32, _L)
    for i in range(_K // _L):
        plsc.store_scatter(sk, [ranks[i]], kvec[i])
        plsc.store_scatter(sl, [ranks[i]], lane + (i * _L))

    bc63 = plsc.load_gather(sk, [jnp.full((_L,), 63, jnp.int32)])

    def body(i, carry):
        b = i * (_U * _L)
        xs = [xv[pl.ds(b + k * _L, _L)] for k in range(_U)]
        pos = [jnp.where(bc63 < xs[k], 64, 0) for k in range(_U)]
        for step in (32, 16, 8, 4, 2, 1):
            for k in range(_U):
                kk = plsc.load_gather(sk, [pos[k] + (step - 1)])
                pos[k] = pos[k] + jnp.where(kk < xs[k], step, 0)
        for k in range(_U):
            ov[pl.ds(b + k * _L, _L)] = plsc.load_gather(sl, [pos[k]])
        return carry

    lax.fori_loop(0, _CH // (_U * _L), body, 0)
    pltpu.sync_copy(ov, out_hbm.at[pl.ds(base, _CH)])


def kernel(x, condition_tensors):
    return _sc_lookup(x, condition_tensors.reshape(_K))


# speedup vs baseline: 1.0376x; 1.0376x over previous
import functools

import jax
import jax.numpy as jnp
from jax import lax
from jax.experimental import pallas as pl
from jax.experimental.pallas import tpu as pltpu
from jax.experimental.pallas import tpu_sc as plsc

_NC = 2
_NS = 16
_NW = _NC * _NS
_L = 16
_K = 128

_N = 500000
_U = 8
_CH = 15744

_M8 = jnp.uint32(0x00FF00FF)


def _bswap_u(v):
    u = plsc.bitcast(v, jnp.uint32)
    t = (u >> 16) | (u << 16)
    return ((t & _M8) << 8) | ((t >> 8) & _M8)


@functools.partial(
    pl.kernel,
    out_type=jax.ShapeDtypeStruct((_N,), jnp.int32),
    mesh=plsc.VectorSubcoreMesh(core_axis_name="c", subcore_axis_name="s"),
    compiler_params=pltpu.CompilerParams(needs_layout_passes=False),
    scratch_types=[
        pltpu.VMEM((_CH,), jnp.int32),
        pltpu.VMEM((_CH,), jnp.int32),
        pltpu.VMEM((_K,), jnp.int32),
    ],
)
def _sc_lookup(x_hbm, keys_hbm, out_hbm, xv, ov, sk):
    wid = lax.axis_index("s") * _NC + lax.axis_index("c")
    base = jnp.minimum(wid * _CH, _N - _CH)
    pltpu.sync_copy(keys_hbm, sk)
    pltpu.sync_copy(x_hbm.at[pl.ds(base, _CH)], xv)

    for j in range(_K // _L):
        s = pl.ds(j * _L, _L)
        sk[s] = plsc.bitcast(_bswap_u(sk[s]), jnp.int32)

    bc63 = plsc.bitcast(
        plsc.load_gather(sk, [jnp.full((_L,), 63, jnp.int32)]), jnp.uint32
    )

    def body(i, carry):
        b = i * (_U * _L)
        xs = [_bswap_u(xv[pl.ds(b + k * _L, _L)]) for k in range(_U)]
        pos = [jnp.where(bc63 < xs[k], 64, 0) for k in range(_U)]
        for step in (32, 16, 8, 4, 2, 1):
            for k in range(_U):
                kk = plsc.load_gather(sk, [pos[k] + (step - 1)])
                lt = plsc.bitcast(kk, jnp.uint32) < xs[k]
                pos[k] = pos[k] + jnp.where(lt, step, 0)
        for k in range(_U):
            ov[pl.ds(b + k * _L, _L)] = pos[k]
        return carry

    lax.fori_loop(0, _CH // (_U * _L), body, 0)
    pltpu.sync_copy(ov, out_hbm.at[pl.ds(base, _CH)])


def kernel(x, condition_tensors):
    return _sc_lookup(x, condition_tensors.reshape(_K))


# speedup vs baseline: 1.0466x; 1.0087x over previous
import functools

import jax
import jax.numpy as jnp
from jax import lax
from jax.experimental import pallas as pl
from jax.experimental.pallas import tpu as pltpu
from jax.experimental.pallas import tpu_sc as plsc

_NC = 2
_NS = 16
_NW = _NC * _NS
_L = 16
_K = 128

_N = 500000
_U = 8
_CH = 15744

_M8 = jnp.uint32(0x00FF00FF)
_SIGN = jnp.uint32(0x80000000)


def _ord32(v):
    u = plsc.bitcast(v, jnp.uint32)
    t = (u >> 16) | (u << 16)
    b = ((t & _M8) << 8) | ((t >> 8) & _M8)
    return plsc.bitcast(b ^ _SIGN, jnp.int32)


@functools.partial(
    pl.kernel,
    out_type=jax.ShapeDtypeStruct((_N,), jnp.int32),
    mesh=plsc.VectorSubcoreMesh(core_axis_name="c", subcore_axis_name="s"),
    compiler_params=pltpu.CompilerParams(needs_layout_passes=False),
    scratch_types=[
        pltpu.VMEM((_CH,), jnp.int32),
        pltpu.VMEM((_CH,), jnp.int32),
        pltpu.VMEM((_K,), jnp.int32),
    ],
)
def _sc_lookup(x_hbm, keys_hbm, out_hbm, xv, ov, sk):
    wid = lax.axis_index("s") * _NC + lax.axis_index("c")
    base = jnp.minimum(wid * _CH, _N - _CH)
    pltpu.sync_copy(keys_hbm, sk)
    pltpu.sync_copy(x_hbm.at[pl.ds(base, _CH)], xv)

    for j in range(_K // _L):
        s = pl.ds(j * _L, _L)
        sk[s] = _ord32(sk[s])

    bc63 = plsc.load_gather(sk, [jnp.full((_L,), 63, jnp.int32)])

    def body(i, carry):
        b = i * (_U * _L)
        xs = [_ord32(xv[pl.ds(b + k * _L, _L)]) for k in range(_U)]
        pos = [jnp.where(bc63 < xs[k], 64, 0) for k in range(_U)]
        for step in (32, 16, 8, 4, 2, 1):
            for k in range(_U):
                kk = plsc.load_gather(sk, [pos[k] + (step - 1)])
                pos[k] = pos[k] + jnp.where(kk < xs[k], step, 0)
        for k in range(_U):
            ov[pl.ds(b + k * _L, _L)] = pos[k]
        return carry

    lax.fori_loop(0, _CH // (_U * _L), body, 0)
    pltpu.sync_copy(ov, out_hbm.at[pl.ds(base, _CH)])


def kernel(x, condition_tensors):
    return _sc_lookup(x, condition_tensors.reshape(_K))


# speedup vs baseline: 1.1146x; 1.0649x over previous
import functools

import jax
import jax.numpy as jnp
from jax import lax
from jax.experimental import pallas as pl
from jax.experimental.pallas import tpu as pltpu
from jax.experimental.pallas import tpu_sc as plsc

_NC = 2
_NS = 16
_NW = _NC * _NS
_L = 16
_K = 128

_N = 500000
_U = 16
_CH = 15872

_SIGN = jnp.int32(-2147483648)


def _ord32(v):
    b0 = jnp.left_shift(jnp.bitwise_and(v, 0xFF), 24)
    b1 = jnp.left_shift(jnp.bitwise_and(v, 0xFF00), 8)
    b2 = jnp.bitwise_and(lax.shift_right_logical(v, 8), 0xFF00)
    b3 = jnp.bitwise_and(lax.shift_right_logical(v, 24), 0xFF)
    return jnp.bitwise_xor(b0 | b1 | b2 | b3, _SIGN)


@functools.partial(
    pl.kernel,
    out_type=jax.ShapeDtypeStruct((_N,), jnp.int32),
    mesh=plsc.VectorSubcoreMesh(core_axis_name="c", subcore_axis_name="s"),
    compiler_params=pltpu.CompilerParams(needs_layout_passes=False),
    scratch_types=[
        pltpu.VMEM((_CH,), jnp.int32),
        pltpu.VMEM((_CH,), jnp.int32),
        pltpu.VMEM((_K,), jnp.int32),
    ],
)
def _sc_lookup(x_hbm, keys_hbm, out_hbm, xv, ov, sk):
    wid = lax.axis_index("s") * _NC + lax.axis_index("c")
    base = jnp.minimum(wid * _CH, _N - _CH)
    pltpu.sync_copy(keys_hbm, sk)
    pltpu.sync_copy(x_hbm.at[pl.ds(base, _CH)], xv)

    for j in range(_K // _L):
        s = pl.ds(j * _L, _L)
        sk[s] = _ord32(sk[s])

    def body(i, carry):
        b = i * (_U * _L)
        xs = [_ord32(xv[pl.ds(b + k * _L, _L)]) for k in range(_U)]
        pos = [jnp.zeros((_L,), jnp.int32) for _ in range(_U)]
        for step in (64, 32, 16, 8, 4, 2, 1):
            for k in range(_U):
                kk = plsc.load_gather(sk, [pos[k] + (step - 1)])
                pos[k] = pos[k] + jnp.where(kk < xs[k], step, 0)
        for k in range(_U):
            ov[pl.ds(b + k * _L, _L)] = pos[k]
        return carry

    lax.fori_loop(0, _CH // (_U * _L), body, 0)
    pltpu.sync_copy(ov, out_hbm.at[pl.ds(base, _CH)])


def kernel(x, condition_tensors):
    return _sc_lookup(x, condition_tensors.reshape(_K))
